# baseline (speedup 1.0000x reference)
import functools

import jax
import jax.numpy as jnp
from jax import lax
from jax.experimental import pallas as pl
from jax.experimental.pallas import tpu as pltpu
from jax.experimental.pallas import tpu_sc as plsc

N = 50000
E = 800000
BATCHES = 16

NUM_CORES = 2
NUM_SUBCORES = 16
SUB = 100
KROWS = 10
EROWS = E // SUB
G = EROWS // (NUM_SUBCORES * KROWS)

ZCHUNK = 3128
ZLAST = N - (NUM_SUBCORES - 1) * ZCHUNK

BLK = 2000
T = N // BLK

_mesh = functools.partial(
    plsc.VectorSubcoreMesh,
    core_axis_name="c", subcore_axis_name="s",
    num_cores=NUM_CORES, num_subcores=NUM_SUBCORES,
)


NBUF = 6
PF = 3


def _sc_edge_loop(x_hbm, e_hbm, acc, idx_bufs, rows_bufs,
                  gsem, ssem, isem, row0, n_outer, col0=None, ncol=None):
    (sA, dA), (sB, dB) = idx_bufs

    def fire_idx(i, sbuf, dbuf):
        r = row0 + i * KROWS
        pltpu.async_copy(e_hbm.at[0, pl.ds(r, KROWS)], sbuf, isem)
        pltpu.async_copy(e_hbm.at[1, pl.ds(r, KROWS)], dbuf, isem)

    def wait_idx(sbuf, dbuf):
        pltpu.make_async_copy(e_hbm.at[0, pl.ds(0, KROWS)], sbuf, isem).wait()
        pltpu.make_async_copy(e_hbm.at[1, pl.ds(0, KROWS)], dbuf, isem).wait()

    def gsrc(idx):
        if col0 is None:
            return x_hbm.at[idx]
        return x_hbm.at[idx, pl.ds(col0, ncol)]

    def process(idx_s, idx_d):
        gd = {}
        sd = {}
        for j in range(PF):
            gd[j] = pltpu.async_copy(gsrc(idx_s.at[j]),
                                     rows_bufs[j % NBUF], gsem)
        for j in range(KROWS):
            nj = j + PF
            if nj < KROWS:
                if nj >= NBUF:
                    sd[nj - NBUF].wait()
                gd[nj] = pltpu.async_copy(gsrc(idx_s.at[nj]),
                                          rows_bufs[nj % NBUF], gsem)
            gd[j].wait()
            sd[j] = pltpu.async_copy(rows_bufs[j % NBUF],
                                     acc.at[idx_d.at[j]], ssem, add=True)
        for j in range(max(0, KROWS - NBUF), KROWS):
            sd[j].wait()

    fire_idx(0, sA, dA)

    def outer(io, _):
        i0 = 2 * io

        @pl.when(i0 + 1 < n_outer)
        def _():
            fire_idx(i0 + 1, sB, dB)

        wait_idx(sA, dA)
        process(sA, dA)

        @pl.when(i0 + 2 < n_outer)
        def _():
            fire_idx(i0 + 2, sA, dA)

        @pl.when(i0 + 1 < n_outer)
        def _():
            wait_idx(sB, dB)
            process(sB, dB)

        return 0

    lax.fori_loop(0, (n_outer + 1) // 2, outer, 0, unroll=False)


def _zero_acc(z_hbm, acc, s):
    @pl.when(s < NUM_SUBCORES - 1)
    def _():
        pltpu.sync_copy(z_hbm, acc.at[pl.ds(s * ZCHUNK, ZCHUNK)])

    @pl.when(s == NUM_SUBCORES - 1)
    def _():
        pltpu.sync_copy(z_hbm.at[pl.ds(0, ZLAST)],
                        acc.at[pl.ds((NUM_SUBCORES - 1) * ZCHUNK, ZLAST)])


def _write_out(acc, out_hbm, c, s, dc):
    @pl.when(s < NUM_SUBCORES - 1)
    def _():
        pltpu.sync_copy(acc.at[pl.ds(s * ZCHUNK, ZCHUNK)],
                        out_hbm.at[pl.ds(s * ZCHUNK, ZCHUNK), pl.ds(dc * c, dc)])

    @pl.when(s == NUM_SUBCORES - 1)
    def _():
        pltpu.sync_copy(
            acc.at[pl.ds((NUM_SUBCORES - 1) * ZCHUNK, ZLAST)],
            out_hbm.at[pl.ds((NUM_SUBCORES - 1) * ZCHUNK, ZLAST),
                       pl.ds(dc * c, dc)])


def _agg1_sc(nodes16, e3, zeros16):
    rows_per_core = EROWS // NUM_CORES
    rows_per_tile = rows_per_core // NUM_SUBCORES
    n_outer = rows_per_tile // KROWS

    def body(nodes_hbm, e_hbm, z_hbm, out_hbm,
             acc, isa, ida, isb, idb, r0, r1, r2, r3, r4, r5,
             gsem, ssem, isem):
        c = lax.axis_index("c")
        s = lax.axis_index("s")
        _zero_acc(z_hbm, acc, s)
        plsc.subcore_barrier()
        row0 = c * rows_per_core + s * rows_per_tile
        _sc_edge_loop(nodes_hbm, e_hbm, acc, [(isa, ida), (isb, idb)],
                      [r0, r1, r2, r3, r4, r5], gsem, ssem, isem,
                      row0, n_outer)
        plsc.subcore_barrier()
        _write_out(acc, out_hbm, c, s, 16)

    k = pl.kernel(
        body,
        out_type=jax.ShapeDtypeStruct((N, 128), jnp.float32),
        mesh=_mesh(),
        compiler_params=pltpu.CompilerParams(use_tc_tiling_on_sc=False),
        scratch_types=[
            pltpu.VMEM_SHARED((N, 16), jnp.float32),
            pltpu.VMEM((KROWS, SUB), jnp.int32),
            pltpu.VMEM((KROWS, SUB), jnp.int32),
            pltpu.VMEM((KROWS, SUB), jnp.int32),
            pltpu.VMEM((KROWS, SUB), jnp.int32),
            pltpu.VMEM((SUB, 16), jnp.float32),
            pltpu.VMEM((SUB, 16), jnp.float32),
            pltpu.VMEM((SUB, 16), jnp.float32),
            pltpu.VMEM((SUB, 16), jnp.float32),
            pltpu.VMEM((SUB, 16), jnp.float32),
            pltpu.VMEM((SUB, 16), jnp.float32),
            pltpu.SemaphoreType.DMA,
            pltpu.SemaphoreType.DMA,
            pltpu.SemaphoreType.DMA,
        ],
    )
    return k(nodes16, e3, zeros16)


def _agg2_sc(x1a, x1b, e3, zeros32):
    rows_per_tile = EROWS // NUM_SUBCORES

    def body(xa_hbm, xb_hbm, e_hbm, z_hbm, out_hbm,
             acc, isa, ida, isb, idb, r0, r1, r2, r3, r4, r5,
             gsem, ssem, isem):
        c = lax.axis_index("c")
        s = lax.axis_index("s")
        _zero_acc(z_hbm, acc, s)
        plsc.subcore_barrier()
        ibufs = [(isa, ida), (isb, idb)]
        bufs = [r0, r1, r2, r3, r4, r5]
        row0 = s * rows_per_tile

        @pl.when(c == 0)
        def _():
            _sc_edge_loop(xa_hbm, e_hbm, acc, ibufs, bufs,
                          gsem, ssem, isem, row0, G)

        @pl.when(c == 1)
        def _():
            _sc_edge_loop(xb_hbm, e_hbm, acc, ibufs, bufs,
                          gsem, ssem, isem, row0, G)

        plsc.subcore_barrier()
        _write_out(acc, out_hbm, c, s, 32)

    k = pl.kernel(
        body,
        out_type=jax.ShapeDtypeStruct((N, 128), jnp.float32),
        mesh=_mesh(),
        compiler_params=pltpu.CompilerParams(use_tc_tiling_on_sc=False),
        scratch_types=[
            pltpu.VMEM_SHARED((N, 32), jnp.float32),
            pltpu.VMEM((KROWS, SUB), jnp.int32),
            pltpu.VMEM((KROWS, SUB), jnp.int32),
            pltpu.VMEM((KROWS, SUB), jnp.int32),
            pltpu.VMEM((KROWS, SUB), jnp.int32),
            pltpu.VMEM((SUB, 32), jnp.float32),
            pltpu.VMEM((SUB, 32), jnp.float32),
            pltpu.VMEM((SUB, 32), jnp.float32),
            pltpu.VMEM((SUB, 32), jnp.float32),
            pltpu.VMEM((SUB, 32), jnp.float32),
            pltpu.VMEM((SUB, 32), jnp.float32),
            pltpu.SemaphoreType.DMA,
            pltpu.SemaphoreType.DMA,
            pltpu.SemaphoreType.DMA,
        ],
    )
    return k(x1a, x1b, e3, zeros32)


def _gelu(x):
    return x * 0.5 * (1.0 + lax.erf(x * (2.0 ** -0.5)))


def _conv1_dense_kernel(x_ref, p_ref, w1_ref, b1_ref, eps_ref, g_ref,
                        be_ref, w2_ref, b2_ref, oa_ref, ob_ref,
                        h_all, ssum, ssq):
    p = pl.program_id(0)
    i = pl.program_id(1)

    @pl.when(p == 0)
    def _():
        x = jnp.pad(x_ref[...], ((0, 0), (0, 11)))
        pp = p_ref[...]
        agg = pp[:, :16] + pp[:, 16:32]
        out = (1.0 + eps_ref[0, 0]) * x + agg
        h = jnp.dot(out, w1_ref[...],
                    preferred_element_type=jnp.float32) + b1_ref[...]
        h_all[pl.ds(i * BLK, BLK), :] = h
        s = jnp.sum(h, axis=0, keepdims=True)
        q = jnp.sum(h * h, axis=0, keepdims=True)

        @pl.when(i == 0)
        def _():
            ssum[...] = s
            ssq[...] = q

        @pl.when(i > 0)
        def _():
            ssum[...] += s
            ssq[...] += q

    @pl.when(p == 1)
    def _():
        m = ssum[...] / N
        v = ssq[...] / N - m * m
        scale = g_ref[...] * lax.rsqrt(v + 1e-5)
        h = h_all[pl.ds(i * BLK, BLK), :]
        xn = (h - m) * scale + be_ref[...]
        ge = _gelu(xn)
        y = jnp.dot(ge, w2_ref[...],
                    preferred_element_type=jnp.float32) + b2_ref[...]
        oa_ref[...] = y[:, :32]
        ob_ref[...] = y[:, 32:]


def _conv1_dense(nodes16, partials, w1, b1, eps, g, be, w2, b2):
    blk = lambda p, i: (i * (1 - p), 0)
    return pl.pallas_call(
        _conv1_dense_kernel,
        grid=(2, T),
        in_specs=[
            pl.BlockSpec((BLK, 5), blk),
            pl.BlockSpec((BLK, 128), lambda p, i: (i * (1 - p), 0)),
            pl.BlockSpec((16, 64), lambda p, i: (0, 0)),
            pl.BlockSpec((1, 64), lambda p, i: (0, 0)),
            pl.BlockSpec((1, 1), lambda p, i: (0, 0)),
            pl.BlockSpec((1, 64), lambda p, i: (0, 0)),
            pl.BlockSpec((1, 64), lambda p, i: (0, 0)),
            pl.BlockSpec((64, 64), lambda p, i: (0, 0)),
            pl.BlockSpec((1, 64), lambda p, i: (0, 0)),
        ],
        out_specs=[
            pl.BlockSpec((BLK, 32), lambda p, i: (i * p, 0)),
            pl.BlockSpec((BLK, 32), lambda p, i: (i * p, 0)),
        ],
        out_shape=[
            jax.ShapeDtypeStruct((N, 32), jnp.float32),
            jax.ShapeDtypeStruct((N, 32), jnp.float32),
        ],
        scratch_shapes=[
            pltpu.VMEM((N, 64), jnp.float32),
            pltpu.VMEM((1, 64), jnp.float32),
            pltpu.VMEM((1, 64), jnp.float32),
        ],
    )(nodes16, partials, w1, b1, eps, g, be, w2, b2)


def _conv2_tail_kernel(xa_ref, xb_ref, p_ref, w3_ref, b3_ref, eps_ref,
                       g_ref, be_ref, w4_ref, b4_ref, oh_ref, w5_ref,
                       b5_ref, out_ref, h_all, ssum, ssq, pacc):
    p = pl.program_id(0)
    i = pl.program_id(1)

    @pl.when(p == 0)
    def _():
        x = jnp.concatenate([xa_ref[...], xb_ref[...]], axis=1)
        agg = p_ref[:, :64]
        out = (1.0 + eps_ref[0, 0]) * x + agg
        h = jnp.dot(out, w3_ref[...],
                    preferred_element_type=jnp.float32) + b3_ref[...]
        h_all[pl.ds(i * BLK, BLK), :] = h
        s = jnp.sum(h, axis=0, keepdims=True)
        q = jnp.sum(h * h, axis=0, keepdims=True)

        @pl.when(i == 0)
        def _():
            ssum[...] = s
            ssq[...] = q

        @pl.when(i > 0)
        def _():
            ssum[...] += s
            ssq[...] += q

    @pl.when(p == 1)
    def _():
        m = ssum[...] / N
        v = ssq[...] / N - m * m
        scale = g_ref[...] * lax.rsqrt(v + 1e-5)
        h = h_all[pl.ds(i * BLK, BLK), :]
        xn = (h - m) * scale + be_ref[...]
        ge = _gelu(xn)
        x2 = jnp.dot(ge, w4_ref[...],
                     preferred_element_type=jnp.float32) + b4_ref[...]
        part = lax.dot_general(oh_ref[...].astype(jnp.float32), x2,
                               (((0,), (0,)), ((), ())),
                               preferred_element_type=jnp.float32)

        @pl.when(i == 0)
        def _():
            pacc[...] = part

        @pl.when(i > 0)
        def _():
            pacc[...] += part

        @pl.when(i == T - 1)
        def _():
            o = jnp.dot(pacc[...], w5_ref[...],
                        preferred_element_type=jnp.float32) + b5_ref[...]
            out_ref[...] = _gelu(o)


def _conv2_tail(x1a, x1b, agg, w3, b3, eps, g, be, w4, b4, onehot, w5, b5):
    return pl.pallas_call(
        _conv2_tail_kernel,
        grid=(2, T),
        in_specs=[
            pl.BlockSpec((BLK, 32), lambda p, i: (i * (1 - p), 0)),
            pl.BlockSpec((BLK, 32), lambda p, i: (i * (1 - p), 0)),
            pl.BlockSpec((BLK, 128), lambda p, i: (i * (1 - p), 0)),
            pl.BlockSpec((64, 128), lambda p, i: (0, 0)),
            pl.BlockSpec((1, 128), lambda p, i: (0, 0)),
            pl.BlockSpec((1, 1), lambda p, i: (0, 0)),
            pl.BlockSpec((1, 128), lambda p, i: (0, 0)),
            pl.BlockSpec((1, 128), lambda p, i: (0, 0)),
            pl.BlockSpec((128, 128), lambda p, i: (0, 0)),
            pl.BlockSpec((1, 128), lambda p, i: (0, 0)),
            pl.BlockSpec((BLK, BATCHES), lambda p, i: (i * p, 0)),
            pl.BlockSpec((128, 128), lambda p, i: (0, 0)),
            pl.BlockSpec((1, 128), lambda p, i: (0, 0)),
        ],
        out_specs=pl.BlockSpec((BATCHES, 128), lambda p, i: (0, 0)),
        out_shape=jax.ShapeDtypeStruct((BATCHES, 128), jnp.float32),
        scratch_shapes=[
            pltpu.VMEM((N, 128), jnp.float32),
            pltpu.VMEM((1, 128), jnp.float32),
            pltpu.VMEM((1, 128), jnp.float32),
            pltpu.VMEM((BATCHES, 128), jnp.float32),
        ],
    )(x1a, x1b, agg, w3, b3, eps, g, be, w4, b4, onehot, w5, b5)


def kernel(nodes, edge_idx, batch_idx, B, eps1, W1, b1, g1, be1, W2, b2,
           eps2, W3, b3, g2, be2, W4, b4, W5, b5):
    nodes16 = jnp.pad(nodes, ((0, 0), (0, 11)))
    W1p = jnp.pad(W1, ((0, 11), (0, 0)))
    e3 = edge_idx.reshape(2, EROWS, SUB)
    zeros16 = jnp.zeros((ZCHUNK, 16), jnp.float32)
    zeros32 = jnp.zeros((ZCHUNK, 32), jnp.float32)
    onehot = (batch_idx[:, None]
              == jnp.arange(BATCHES, dtype=jnp.int32)[None, :]
              ).astype(jnp.bfloat16)

    eps1v = jnp.reshape(eps1, (1, 1))
    eps2v = jnp.reshape(eps2, (1, 1))
    row = lambda a: jnp.reshape(a, (1, -1))

    p1 = _agg1_sc(nodes16, e3, zeros16)
    x1a, x1b = _conv1_dense(nodes, p1, W1p, row(b1), eps1v,
                            row(g1), row(be1), W2, row(b2))

    agg2 = _agg2_sc(x1a, x1b, e3, zeros32)
    out = _conv2_tail(x1a, x1b, agg2, W3, row(b3), eps2v, row(g2), row(be2),
                      W4, row(b4), onehot, W5, row(b5))
    return out


# speedup vs baseline: 1.0538x; 1.0538x over previous
import functools

import jax
import jax.numpy as jnp
from jax import lax
from jax.experimental import pallas as pl
from jax.experimental.pallas import tpu as pltpu
from jax.experimental.pallas import tpu_sc as plsc

N = 50000
E = 800000
BATCHES = 16

NUM_CORES = 2
NUM_SUBCORES = 16
SUB = 100
KROWS = 10
EROWS = E // SUB
G = EROWS // (NUM_SUBCORES * KROWS)

ZCHUNK = 3128
ZLAST = N - (NUM_SUBCORES - 1) * ZCHUNK

BLK = 2000
T = N // BLK

_mesh = functools.partial(
    plsc.VectorSubcoreMesh,
    core_axis_name="c", subcore_axis_name="s",
    num_cores=NUM_CORES, num_subcores=NUM_SUBCORES,
)


NBUF = 8
PF = 4


def _sc_edge_loop(x_hbm, e_hbm, acc, idx_bufs, rows_bufs,
                  gsem, ssem, isem, row0, n_outer, col0=None, ncol=None):
    (sA, dA), (sB, dB) = idx_bufs

    def fire_idx(i, sbuf, dbuf):
        r = row0 + i * KROWS
        pltpu.async_copy(e_hbm.at[0, pl.ds(r, KROWS)], sbuf, isem)
        pltpu.async_copy(e_hbm.at[1, pl.ds(r, KROWS)], dbuf, isem)

    def wait_idx(sbuf, dbuf):
        pltpu.make_async_copy(e_hbm.at[0, pl.ds(0, KROWS)], sbuf, isem).wait()
        pltpu.make_async_copy(e_hbm.at[1, pl.ds(0, KROWS)], dbuf, isem).wait()

    def gsrc(idx):
        if col0 is None:
            return x_hbm.at[idx]
        return x_hbm.at[idx, pl.ds(col0, ncol)]

    def process(idx_s, idx_d):
        gd = {}
        sd = {}
        for j in range(PF):
            gd[j] = pltpu.async_copy(gsrc(idx_s.at[j]),
                                     rows_bufs[j % NBUF], gsem)
        for j in range(KROWS):
            nj = j + PF
            if nj < KROWS:
                if nj >= NBUF:
                    sd[nj - NBUF].wait()
                gd[nj] = pltpu.async_copy(gsrc(idx_s.at[nj]),
                                          rows_bufs[nj % NBUF], gsem)
            gd[j].wait()
            sd[j] = pltpu.async_copy(rows_bufs[j % NBUF],
                                     acc.at[idx_d.at[j]], ssem, add=True)
        for j in range(max(0, KROWS - NBUF), KROWS):
            sd[j].wait()

    fire_idx(0, sA, dA)

    def outer(io, _):
        i0 = 2 * io

        @pl.when(i0 + 1 < n_outer)
        def _():
            fire_idx(i0 + 1, sB, dB)

        wait_idx(sA, dA)
        process(sA, dA)

        @pl.when(i0 + 2 < n_outer)
        def _():
            fire_idx(i0 + 2, sA, dA)

        @pl.when(i0 + 1 < n_outer)
        def _():
            wait_idx(sB, dB)
            process(sB, dB)

        return 0

    lax.fori_loop(0, (n_outer + 1) // 2, outer, 0, unroll=False)


def _zero_acc(z_hbm, acc, s):
    @pl.when(s < NUM_SUBCORES - 1)
    def _():
        pltpu.sync_copy(z_hbm, acc.at[pl.ds(s * ZCHUNK, ZCHUNK)])

    @pl.when(s == NUM_SUBCORES - 1)
    def _():
        pltpu.sync_copy(z_hbm.at[pl.ds(0, ZLAST)],
                        acc.at[pl.ds((NUM_SUBCORES - 1) * ZCHUNK, ZLAST)])


def _write_out(acc, out_hbm, c, s, dc):
    @pl.when(s < NUM_SUBCORES - 1)
    def _():
        pltpu.sync_copy(acc.at[pl.ds(s * ZCHUNK, ZCHUNK)],
                        out_hbm.at[pl.ds(s * ZCHUNK, ZCHUNK), pl.ds(dc * c, dc)])

    @pl.when(s == NUM_SUBCORES - 1)
    def _():
        pltpu.sync_copy(
            acc.at[pl.ds((NUM_SUBCORES - 1) * ZCHUNK, ZLAST)],
            out_hbm.at[pl.ds((NUM_SUBCORES - 1) * ZCHUNK, ZLAST),
                       pl.ds(dc * c, dc)])


def _agg1_sc(nodes16, e3, zeros16):
    rows_per_core = EROWS // NUM_CORES
    rows_per_tile = rows_per_core // NUM_SUBCORES
    n_outer = rows_per_tile // KROWS

    def body(nodes_hbm, e_hbm, z_hbm, out_hbm,
             acc, isa, ida, isb, idb, r0, r1, r2, r3, r4, r5, r6, r7,
             gsem, ssem, isem):
        c = lax.axis_index("c")
        s = lax.axis_index("s")
        _zero_acc(z_hbm, acc, s)
        plsc.subcore_barrier()
        row0 = c * rows_per_core + s * rows_per_tile
        _sc_edge_loop(nodes_hbm, e_hbm, acc, [(isa, ida), (isb, idb)],
                      [r0, r1, r2, r3, r4, r5, r6, r7], gsem, ssem, isem,
                      row0, n_outer)
        plsc.subcore_barrier()
        _write_out(acc, out_hbm, c, s, 16)

    k = pl.kernel(
        body,
        out_type=jax.ShapeDtypeStruct((N, 128), jnp.float32),
        mesh=_mesh(),
        compiler_params=pltpu.CompilerParams(use_tc_tiling_on_sc=False),
        scratch_types=[
            pltpu.VMEM_SHARED((N, 16), jnp.float32),
            pltpu.VMEM((KROWS, SUB), jnp.int32),
            pltpu.VMEM((KROWS, SUB), jnp.int32),
            pltpu.VMEM((KROWS, SUB), jnp.int32),
            pltpu.VMEM((KROWS, SUB), jnp.int32),
            pltpu.VMEM((SUB, 16), jnp.float32),
            pltpu.VMEM((SUB, 16), jnp.float32),
            pltpu.VMEM((SUB, 16), jnp.float32),
            pltpu.VMEM((SUB, 16), jnp.float32),
            pltpu.VMEM((SUB, 16), jnp.float32),
            pltpu.VMEM((SUB, 16), jnp.float32),
            pltpu.VMEM((SUB, 16), jnp.float32),
            pltpu.VMEM((SUB, 16), jnp.float32),
            pltpu.SemaphoreType.DMA,
            pltpu.SemaphoreType.DMA,
            pltpu.SemaphoreType.DMA,
        ],
    )
    return k(nodes16, e3, zeros16)


def _agg2_sc(x1a, x1b, e3, zeros32):
    rows_per_tile = EROWS // NUM_SUBCORES

    def body(xa_hbm, xb_hbm, e_hbm, z_hbm, out_hbm,
             acc, isa, ida, isb, idb, r0, r1, r2, r3, r4, r5, r6, r7,
             gsem, ssem, isem):
        c = lax.axis_index("c")
        s = lax.axis_index("s")
        _zero_acc(z_hbm, acc, s)
        plsc.subcore_barrier()
        ibufs = [(isa, ida), (isb, idb)]
        bufs = [r0, r1, r2, r3, r4, r5, r6, r7]
        row0 = s * rows_per_tile

        @pl.when(c == 0)
        def _():
            _sc_edge_loop(xa_hbm, e_hbm, acc, ibufs, bufs,
                          gsem, ssem, isem, row0, G)

        @pl.when(c == 1)
        def _():
            _sc_edge_loop(xb_hbm, e_hbm, acc, ibufs, bufs,
                          gsem, ssem, isem, row0, G)

        plsc.subcore_barrier()
        _write_out(acc, out_hbm, c, s, 32)

    k = pl.kernel(
        body,
        out_type=jax.ShapeDtypeStruct((N, 128), jnp.float32),
        mesh=_mesh(),
        compiler_params=pltpu.CompilerParams(use_tc_tiling_on_sc=False),
        scratch_types=[
            pltpu.VMEM_SHARED((N, 32), jnp.float32),
            pltpu.VMEM((KROWS, SUB), jnp.int32),
            pltpu.VMEM((KROWS, SUB), jnp.int32),
            pltpu.VMEM((KROWS, SUB), jnp.int32),
            pltpu.VMEM((KROWS, SUB), jnp.int32),
            pltpu.VMEM((SUB, 32), jnp.float32),
            pltpu.VMEM((SUB, 32), jnp.float32),
            pltpu.VMEM((SUB, 32), jnp.float32),
            pltpu.VMEM((SUB, 32), jnp.float32),
            pltpu.VMEM((SUB, 32), jnp.float32),
            pltpu.VMEM((SUB, 32), jnp.float32),
            pltpu.VMEM((SUB, 32), jnp.float32),
            pltpu.VMEM((SUB, 32), jnp.float32),
            pltpu.SemaphoreType.DMA,
            pltpu.SemaphoreType.DMA,
            pltpu.SemaphoreType.DMA,
        ],
    )
    return k(x1a, x1b, e3, zeros32)


def _gelu(x):
    return x * 0.5 * (1.0 + lax.erf(x * (2.0 ** -0.5)))


def _conv1_dense_kernel(x_ref, p_ref, w1_ref, b1_ref, eps_ref, g_ref,
                        be_ref, w2_ref, b2_ref, oa_ref, ob_ref,
                        h_all, ssum, ssq):
    p = pl.program_id(0)
    i = pl.program_id(1)

    @pl.when(p == 0)
    def _():
        x = x_ref[...]
        pp = p_ref[...]
        agg = pp[:, :16] + pp[:, 16:32]
        out = (1.0 + eps_ref[0, 0]) * x + agg
        h = jnp.dot(out, w1_ref[...],
                    preferred_element_type=jnp.float32) + b1_ref[...]
        h_all[pl.ds(i * BLK, BLK), :] = h
        s = jnp.sum(h, axis=0, keepdims=True)
        q = jnp.sum(h * h, axis=0, keepdims=True)

        @pl.when(i == 0)
        def _():
            ssum[...] = s
            ssq[...] = q

        @pl.when(i > 0)
        def _():
            ssum[...] += s
            ssq[...] += q

    @pl.when(p == 1)
    def _():
        m = ssum[...] / N
        v = ssq[...] / N - m * m
        scale = g_ref[...] * lax.rsqrt(v + 1e-5)
        h = h_all[pl.ds(i * BLK, BLK), :]
        xn = (h - m) * scale + be_ref[...]
        ge = _gelu(xn)
        y = jnp.dot(ge, w2_ref[...],
                    preferred_element_type=jnp.float32) + b2_ref[...]
        oa_ref[...] = y[:, :32]
        ob_ref[...] = y[:, 32:]


def _conv1_dense(nodes16, partials, w1, b1, eps, g, be, w2, b2):
    blk = lambda p, i: (i * (1 - p), 0)
    return pl.pallas_call(
        _conv1_dense_kernel,
        grid=(2, T),
        in_specs=[
            pl.BlockSpec((BLK, 16), blk),
            pl.BlockSpec((BLK, 128), lambda p, i: (i * (1 - p), 0)),
            pl.BlockSpec((16, 64), lambda p, i: (0, 0)),
            pl.BlockSpec((1, 64), lambda p, i: (0, 0)),
            pl.BlockSpec((1, 1), lambda p, i: (0, 0)),
            pl.BlockSpec((1, 64), lambda p, i: (0, 0)),
            pl.BlockSpec((1, 64), lambda p, i: (0, 0)),
            pl.BlockSpec((64, 64), lambda p, i: (0, 0)),
            pl.BlockSpec((1, 64), lambda p, i: (0, 0)),
        ],
        out_specs=[
            pl.BlockSpec((BLK, 32), lambda p, i: (i * p, 0)),
            pl.BlockSpec((BLK, 32), lambda p, i: (i * p, 0)),
        ],
        out_shape=[
            jax.ShapeDtypeStruct((N, 32), jnp.float32),
            jax.ShapeDtypeStruct((N, 32), jnp.float32),
        ],
        scratch_shapes=[
            pltpu.VMEM((N, 64), jnp.float32),
            pltpu.VMEM((1, 64), jnp.float32),
            pltpu.VMEM((1, 64), jnp.float32),
        ],
    )(nodes16, partials, w1, b1, eps, g, be, w2, b2)


def _conv2_tail_kernel(xa_ref, xb_ref, p_ref, w3_ref, b3_ref, eps_ref,
                       g_ref, be_ref, w4_ref, b4_ref, oh_ref, w5_ref,
                       b5_ref, out_ref, h_all, ssum, ssq, pacc):
    p = pl.program_id(0)
    i = pl.program_id(1)

    @pl.when(p == 0)
    def _():
        x = jnp.concatenate([xa_ref[...], xb_ref[...]], axis=1)
        agg = p_ref[:, :64]
        out = (1.0 + eps_ref[0, 0]) * x + agg
        h = jnp.dot(out, w3_ref[...],
                    preferred_element_type=jnp.float32) + b3_ref[...]
        h_all[pl.ds(i * BLK, BLK), :] = h
        s = jnp.sum(h, axis=0, keepdims=True)
        q = jnp.sum(h * h, axis=0, keepdims=True)

        @pl.when(i == 0)
        def _():
            ssum[...] = s
            ssq[...] = q

        @pl.when(i > 0)
        def _():
            ssum[...] += s
            ssq[...] += q

    @pl.when(p == 1)
    def _():
        m = ssum[...] / N
        v = ssq[...] / N - m * m
        scale = g_ref[...] * lax.rsqrt(v + 1e-5)
        h = h_all[pl.ds(i * BLK, BLK), :]
        xn = (h - m) * scale + be_ref[...]
        ge = _gelu(xn)
        x2 = jnp.dot(ge, w4_ref[...],
                     preferred_element_type=jnp.float32) + b4_ref[...]
        part = lax.dot_general(oh_ref[...].astype(jnp.float32), x2,
                               (((0,), (0,)), ((), ())),
                               preferred_element_type=jnp.float32)

        @pl.when(i == 0)
        def _():
            pacc[...] = part

        @pl.when(i > 0)
        def _():
            pacc[...] += part

        @pl.when(i == T - 1)
        def _():
            o = jnp.dot(pacc[...], w5_ref[...],
                        preferred_element_type=jnp.float32) + b5_ref[...]
            out_ref[...] = _gelu(o)


def _conv2_tail(x1a, x1b, agg, w3, b3, eps, g, be, w4, b4, onehot, w5, b5):
    return pl.pallas_call(
        _conv2_tail_kernel,
        grid=(2, T),
        in_specs=[
            pl.BlockSpec((BLK, 32), lambda p, i: (i * (1 - p), 0)),
            pl.BlockSpec((BLK, 32), lambda p, i: (i * (1 - p), 0)),
            pl.BlockSpec((BLK, 128), lambda p, i: (i * (1 - p), 0)),
            pl.BlockSpec((64, 128), lambda p, i: (0, 0)),
            pl.BlockSpec((1, 128), lambda p, i: (0, 0)),
            pl.BlockSpec((1, 1), lambda p, i: (0, 0)),
            pl.BlockSpec((1, 128), lambda p, i: (0, 0)),
            pl.BlockSpec((1, 128), lambda p, i: (0, 0)),
            pl.BlockSpec((128, 128), lambda p, i: (0, 0)),
            pl.BlockSpec((1, 128), lambda p, i: (0, 0)),
            pl.BlockSpec((BLK, BATCHES), lambda p, i: (i * p, 0)),
            pl.BlockSpec((128, 128), lambda p, i: (0, 0)),
            pl.BlockSpec((1, 128), lambda p, i: (0, 0)),
        ],
        out_specs=pl.BlockSpec((BATCHES, 128), lambda p, i: (0, 0)),
        out_shape=jax.ShapeDtypeStruct((BATCHES, 128), jnp.float32),
        scratch_shapes=[
            pltpu.VMEM((N, 128), jnp.float32),
            pltpu.VMEM((1, 128), jnp.float32),
            pltpu.VMEM((1, 128), jnp.float32),
            pltpu.VMEM((BATCHES, 128), jnp.float32),
        ],
    )(x1a, x1b, agg, w3, b3, eps, g, be, w4, b4, onehot, w5, b5)


def kernel(nodes, edge_idx, batch_idx, B, eps1, W1, b1, g1, be1, W2, b2,
           eps2, W3, b3, g2, be2, W4, b4, W5, b5):
    nodes16 = jnp.pad(nodes, ((0, 0), (0, 11)))
    W1p = jnp.pad(W1, ((0, 11), (0, 0)))
    e3 = edge_idx.reshape(2, EROWS, SUB)
    zeros16 = jnp.zeros((ZCHUNK, 16), jnp.float32)
    zeros32 = jnp.zeros((ZCHUNK, 32), jnp.float32)
    onehot = (batch_idx[:, None]
              == jnp.arange(BATCHES, dtype=jnp.int32)[None, :]
              ).astype(jnp.bfloat16)

    eps1v = jnp.reshape(eps1, (1, 1))
    eps2v = jnp.reshape(eps2, (1, 1))
    row = lambda a: jnp.reshape(a, (1, -1))

    p1 = _agg1_sc(nodes16, e3, zeros16)
    x1a, x1b = _conv1_dense(nodes16, p1, W1p, row(b1), eps1v,
                            row(g1), row(be1), W2, row(b2))

    agg2 = _agg2_sc(x1a, x1b, e3, zeros32)
    out = _conv2_tail(x1a, x1b, agg2, W3, row(b3), eps2v, row(g2), row(be2),
                      W4, row(b4), onehot, W5, row(b5))
    return out


# speedup vs baseline: 1.0687x; 1.0142x over previous
import functools

import jax
import jax.numpy as jnp
from jax import lax
from jax.experimental import pallas as pl
from jax.experimental.pallas import tpu as pltpu
from jax.experimental.pallas import tpu_sc as plsc

N = 50000
E = 800000
BATCHES = 16

NUM_CORES = 2
NUM_SUBCORES = 16
SUB = 100
KROWS = 10
EROWS = E // SUB
G = EROWS // (NUM_SUBCORES * KROWS)

ZCHUNK = 3128
ZLAST = N - (NUM_SUBCORES - 1) * ZCHUNK

BLK = 2000
T = N // BLK

_mesh = functools.partial(
    plsc.VectorSubcoreMesh,
    core_axis_name="c", subcore_axis_name="s",
    num_cores=NUM_CORES, num_subcores=NUM_SUBCORES,
)


NBUF = 8
PF = 4


def _sc_edge_loop(x_hbm, e_hbm, acc, idx_bufs, rows_bufs,
                  gsem, ssem, isem, row0, n_outer, col0=None, ncol=None,
                  nbuf=NBUF, pf=PF):
    (sA, dA), (sB, dB) = idx_bufs

    def fire_idx(i, sbuf, dbuf):
        r = row0 + i * KROWS
        pltpu.async_copy(e_hbm.at[0, pl.ds(r, KROWS)], sbuf, isem)
        pltpu.async_copy(e_hbm.at[1, pl.ds(r, KROWS)], dbuf, isem)

    def wait_idx(sbuf, dbuf):
        pltpu.make_async_copy(e_hbm.at[0, pl.ds(0, KROWS)], sbuf, isem).wait()
        pltpu.make_async_copy(e_hbm.at[1, pl.ds(0, KROWS)], dbuf, isem).wait()

    def gsrc(idx):
        if col0 is None:
            return x_hbm.at[idx]
        return x_hbm.at[idx, pl.ds(col0, ncol)]

    def process(idx_s, idx_d):
        gd = {}
        sd = {}
        for j in range(pf):
            gd[j] = pltpu.async_copy(gsrc(idx_s.at[j]),
                                     rows_bufs[j % nbuf], gsem)
        for j in range(KROWS):
            nj = j + pf
            if nj < KROWS:
                if nj >= nbuf:
                    sd[nj - nbuf].wait()
                gd[nj] = pltpu.async_copy(gsrc(idx_s.at[nj]),
                                          rows_bufs[nj % nbuf], gsem)
            gd[j].wait()
            sd[j] = pltpu.async_copy(rows_bufs[j % nbuf],
                                     acc.at[idx_d.at[j]], ssem, add=True)
        for j in range(max(0, KROWS - nbuf), KROWS):
            sd[j].wait()

    fire_idx(0, sA, dA)

    def outer(io, _):
        i0 = 2 * io

        @pl.when(i0 + 1 < n_outer)
        def _():
            fire_idx(i0 + 1, sB, dB)

        wait_idx(sA, dA)
        process(sA, dA)

        @pl.when(i0 + 2 < n_outer)
        def _():
            fire_idx(i0 + 2, sA, dA)

        @pl.when(i0 + 1 < n_outer)
        def _():
            wait_idx(sB, dB)
            process(sB, dB)

        return 0

    lax.fori_loop(0, (n_outer + 1) // 2, outer, 0, unroll=False)


def _zero_acc(z_hbm, acc, s):
    @pl.when(s < NUM_SUBCORES - 1)
    def _():
        pltpu.sync_copy(z_hbm, acc.at[pl.ds(s * ZCHUNK, ZCHUNK)])

    @pl.when(s == NUM_SUBCORES - 1)
    def _():
        pltpu.sync_copy(z_hbm.at[pl.ds(0, ZLAST)],
                        acc.at[pl.ds((NUM_SUBCORES - 1) * ZCHUNK, ZLAST)])


def _write_out(acc, out_hbm, c, s, dc):
    @pl.when(s < NUM_SUBCORES - 1)
    def _():
        pltpu.sync_copy(acc.at[pl.ds(s * ZCHUNK, ZCHUNK)],
                        out_hbm.at[pl.ds(s * ZCHUNK, ZCHUNK), pl.ds(dc * c, dc)])

    @pl.when(s == NUM_SUBCORES - 1)
    def _():
        pltpu.sync_copy(
            acc.at[pl.ds((NUM_SUBCORES - 1) * ZCHUNK, ZLAST)],
            out_hbm.at[pl.ds((NUM_SUBCORES - 1) * ZCHUNK, ZLAST),
                       pl.ds(dc * c, dc)])


def _agg1_sc(nodes16, e3, zeros16):
    rows_per_core = EROWS // NUM_CORES
    rows_per_tile = rows_per_core // NUM_SUBCORES
    n_outer = rows_per_tile // KROWS

    def body(nodes_hbm, e_hbm, z_hbm, out_hbm,
             acc, isa, ida, isb, idb, r0, r1, r2, r3, r4, r5, r6, r7,
             r8, r9, gsem, ssem, isem):
        c = lax.axis_index("c")
        s = lax.axis_index("s")
        _zero_acc(z_hbm, acc, s)
        plsc.subcore_barrier()
        row0 = c * rows_per_core + s * rows_per_tile
        _sc_edge_loop(nodes_hbm, e_hbm, acc, [(isa, ida), (isb, idb)],
                      [r0, r1, r2, r3, r4, r5, r6, r7, r8, r9],
                      gsem, ssem, isem, row0, n_outer, nbuf=10, pf=5)
        plsc.subcore_barrier()
        _write_out(acc, out_hbm, c, s, 16)

    k = pl.kernel(
        body,
        out_type=jax.ShapeDtypeStruct((N, 128), jnp.float32),
        mesh=_mesh(),
        compiler_params=pltpu.CompilerParams(use_tc_tiling_on_sc=False),
        scratch_types=[
            pltpu.VMEM_SHARED((N, 16), jnp.float32),
            pltpu.VMEM((KROWS, SUB), jnp.int32),
            pltpu.VMEM((KROWS, SUB), jnp.int32),
            pltpu.VMEM((KROWS, SUB), jnp.int32),
            pltpu.VMEM((KROWS, SUB), jnp.int32),
            pltpu.VMEM((SUB, 16), jnp.float32),
            pltpu.VMEM((SUB, 16), jnp.float32),
            pltpu.VMEM((SUB, 16), jnp.float32),
            pltpu.VMEM((SUB, 16), jnp.float32),
            pltpu.VMEM((SUB, 16), jnp.float32),
            pltpu.VMEM((SUB, 16), jnp.float32),
            pltpu.VMEM((SUB, 16), jnp.float32),
            pltpu.VMEM((SUB, 16), jnp.float32),
            pltpu.VMEM((SUB, 16), jnp.float32),
            pltpu.VMEM((SUB, 16), jnp.float32),
            pltpu.SemaphoreType.DMA,
            pltpu.SemaphoreType.DMA,
            pltpu.SemaphoreType.DMA,
        ],
    )
    return k(nodes16, e3, zeros16)


def _agg2_sc(x1a, x1b, e3, zeros32):
    rows_per_tile = EROWS // NUM_SUBCORES

    def body(xa_hbm, xb_hbm, e_hbm, z_hbm, out_hbm,
             acc, isa, ida, isb, idb, r0, r1, r2, r3, r4, r5, r6, r7,
             gsem, ssem, isem):
        c = lax.axis_index("c")
        s = lax.axis_index("s")
        _zero_acc(z_hbm, acc, s)
        plsc.subcore_barrier()
        ibufs = [(isa, ida), (isb, idb)]
        bufs = [r0, r1, r2, r3, r4, r5, r6, r7]
        row0 = s * rows_per_tile

        @pl.when(c == 0)
        def _():
            _sc_edge_loop(xa_hbm, e_hbm, acc, ibufs, bufs,
                          gsem, ssem, isem, row0, G)

        @pl.when(c == 1)
        def _():
            _sc_edge_loop(xb_hbm, e_hbm, acc, ibufs, bufs,
                          gsem, ssem, isem, row0, G)

        plsc.subcore_barrier()
        _write_out(acc, out_hbm, c, s, 32)

    k = pl.kernel(
        body,
        out_type=jax.ShapeDtypeStruct((N, 128), jnp.float32),
        mesh=_mesh(),
        compiler_params=pltpu.CompilerParams(use_tc_tiling_on_sc=False),
        scratch_types=[
            pltpu.VMEM_SHARED((N, 32), jnp.float32),
            pltpu.VMEM((KROWS, SUB), jnp.int32),
            pltpu.VMEM((KROWS, SUB), jnp.int32),
            pltpu.VMEM((KROWS, SUB), jnp.int32),
            pltpu.VMEM((KROWS, SUB), jnp.int32),
            pltpu.VMEM((SUB, 32), jnp.float32),
            pltpu.VMEM((SUB, 32), jnp.float32),
            pltpu.VMEM((SUB, 32), jnp.float32),
            pltpu.VMEM((SUB, 32), jnp.float32),
            pltpu.VMEM((SUB, 32), jnp.float32),
            pltpu.VMEM((SUB, 32), jnp.float32),
            pltpu.VMEM((SUB, 32), jnp.float32),
            pltpu.VMEM((SUB, 32), jnp.float32),
            pltpu.SemaphoreType.DMA,
            pltpu.SemaphoreType.DMA,
            pltpu.SemaphoreType.DMA,
        ],
    )
    return k(x1a, x1b, e3, zeros32)


def _gelu(x):
    return x * 0.5 * (1.0 + lax.erf(x * (2.0 ** -0.5)))


def _conv1_dense_kernel(x_ref, p_ref, w1_ref, b1_ref, eps_ref, g_ref,
                        be_ref, w2_ref, b2_ref, oa_ref, ob_ref,
                        h_all, ssum, ssq):
    p = pl.program_id(0)
    i = pl.program_id(1)

    @pl.when(p == 0)
    def _():
        x = x_ref[...]
        pp = p_ref[...]
        agg = pp[:, :16] + pp[:, 16:32]
        out = (1.0 + eps_ref[0, 0]) * x + agg
        h = jnp.dot(out, w1_ref[...],
                    preferred_element_type=jnp.float32) + b1_ref[...]
        h_all[pl.ds(i * BLK, BLK), :] = h
        s = jnp.sum(h, axis=0, keepdims=True)
        q = jnp.sum(h * h, axis=0, keepdims=True)

        @pl.when(i == 0)
        def _():
            ssum[...] = s
            ssq[...] = q

        @pl.when(i > 0)
        def _():
            ssum[...] += s
            ssq[...] += q

    @pl.when(p == 1)
    def _():
        m = ssum[...] / N
        v = ssq[...] / N - m * m
        scale = g_ref[...] * lax.rsqrt(v + 1e-5)
        h = h_all[pl.ds(i * BLK, BLK), :]
        xn = (h - m) * scale + be_ref[...]
        ge = _gelu(xn)
        y = jnp.dot(ge, w2_ref[...],
                    preferred_element_type=jnp.float32) + b2_ref[...]
        oa_ref[...] = y[:, :32]
        ob_ref[...] = y[:, 32:]


def _conv1_dense(nodes16, partials, w1, b1, eps, g, be, w2, b2):
    blk = lambda p, i: (i * (1 - p), 0)
    return pl.pallas_call(
        _conv1_dense_kernel,
        grid=(2, T),
        in_specs=[
            pl.BlockSpec((BLK, 16), blk),
            pl.BlockSpec((BLK, 128), lambda p, i: (i * (1 - p), 0)),
            pl.BlockSpec((16, 64), lambda p, i: (0, 0)),
            pl.BlockSpec((1, 64), lambda p, i: (0, 0)),
            pl.BlockSpec((1, 1), lambda p, i: (0, 0)),
            pl.BlockSpec((1, 64), lambda p, i: (0, 0)),
            pl.BlockSpec((1, 64), lambda p, i: (0, 0)),
            pl.BlockSpec((64, 64), lambda p, i: (0, 0)),
            pl.BlockSpec((1, 64), lambda p, i: (0, 0)),
        ],
        out_specs=[
            pl.BlockSpec((BLK, 32), lambda p, i: (i * p, 0)),
            pl.BlockSpec((BLK, 32), lambda p, i: (i * p, 0)),
        ],
        out_shape=[
            jax.ShapeDtypeStruct((N, 32), jnp.float32),
            jax.ShapeDtypeStruct((N, 32), jnp.float32),
        ],
        scratch_shapes=[
            pltpu.VMEM((N, 64), jnp.float32),
            pltpu.VMEM((1, 64), jnp.float32),
            pltpu.VMEM((1, 64), jnp.float32),
        ],
    )(nodes16, partials, w1, b1, eps, g, be, w2, b2)


def _conv2_tail_kernel(xa_ref, xb_ref, p_ref, w3_ref, b3_ref, eps_ref,
                       g_ref, be_ref, w4_ref, b4_ref, oh_ref, w5_ref,
                       b5_ref, out_ref, h_all, ssum, ssq, pacc):
    p = pl.program_id(0)
    i = pl.program_id(1)

    @pl.when(p == 0)
    def _():
        x = jnp.concatenate([xa_ref[...], xb_ref[...]], axis=1)
        agg = p_ref[:, :64]
        out = (1.0 + eps_ref[0, 0]) * x + agg
        h = jnp.dot(out, w3_ref[...],
                    preferred_element_type=jnp.float32) + b3_ref[...]
        h_all[pl.ds(i * BLK, BLK), :] = h
        s = jnp.sum(h, axis=0, keepdims=True)
        q = jnp.sum(h * h, axis=0, keepdims=True)

        @pl.when(i == 0)
        def _():
            ssum[...] = s
            ssq[...] = q

        @pl.when(i > 0)
        def _():
            ssum[...] += s
            ssq[...] += q

    @pl.when(p == 1)
    def _():
        m = ssum[...] / N
        v = ssq[...] / N - m * m
        scale = g_ref[...] * lax.rsqrt(v + 1e-5)
        h = h_all[pl.ds(i * BLK, BLK), :]
        xn = (h - m) * scale + be_ref[...]
        ge = _gelu(xn)
        x2 = jnp.dot(ge, w4_ref[...],
                     preferred_element_type=jnp.float32) + b4_ref[...]
        part = lax.dot_general(oh_ref[...].astype(jnp.float32), x2,
                               (((0,), (0,)), ((), ())),
                               preferred_element_type=jnp.float32)

        @pl.when(i == 0)
        def _():
            pacc[...] = part

        @pl.when(i > 0)
        def _():
            pacc[...] += part

        @pl.when(i == T - 1)
        def _():
            o = jnp.dot(pacc[...], w5_ref[...],
                        preferred_element_type=jnp.float32) + b5_ref[...]
            out_ref[...] = _gelu(o)


def _conv2_tail(x1a, x1b, agg, w3, b3, eps, g, be, w4, b4, onehot, w5, b5):
    return pl.pallas_call(
        _conv2_tail_kernel,
        grid=(2, T),
        in_specs=[
            pl.BlockSpec((BLK, 32), lambda p, i: (i * (1 - p), 0)),
            pl.BlockSpec((BLK, 32), lambda p, i: (i * (1 - p), 0)),
            pl.BlockSpec((BLK, 128), lambda p, i: (i * (1 - p), 0)),
            pl.BlockSpec((64, 128), lambda p, i: (0, 0)),
            pl.BlockSpec((1, 128), lambda p, i: (0, 0)),
            pl.BlockSpec((1, 1), lambda p, i: (0, 0)),
            pl.BlockSpec((1, 128), lambda p, i: (0, 0)),
            pl.BlockSpec((1, 128), lambda p, i: (0, 0)),
            pl.BlockSpec((128, 128), lambda p, i: (0, 0)),
            pl.BlockSpec((1, 128), lambda p, i: (0, 0)),
            pl.BlockSpec((BLK, BATCHES), lambda p, i: (i * p, 0)),
            pl.BlockSpec((128, 128), lambda p, i: (0, 0)),
            pl.BlockSpec((1, 128), lambda p, i: (0, 0)),
        ],
        out_specs=pl.BlockSpec((BATCHES, 128), lambda p, i: (0, 0)),
        out_shape=jax.ShapeDtypeStruct((BATCHES, 128), jnp.float32),
        scratch_shapes=[
            pltpu.VMEM((N, 128), jnp.float32),
            pltpu.VMEM((1, 128), jnp.float32),
            pltpu.VMEM((1, 128), jnp.float32),
            pltpu.VMEM((BATCHES, 128), jnp.float32),
        ],
    )(x1a, x1b, agg, w3, b3, eps, g, be, w4, b4, onehot, w5, b5)


def kernel(nodes, edge_idx, batch_idx, B, eps1, W1, b1, g1, be1, W2, b2,
           eps2, W3, b3, g2, be2, W4, b4, W5, b5):
    nodes16 = jnp.pad(nodes, ((0, 0), (0, 11)))
    W1p = jnp.pad(W1, ((0, 11), (0, 0)))
    e3 = edge_idx.reshape(2, EROWS, SUB)
    zeros16 = jnp.zeros((ZCHUNK, 16), jnp.float32)
    zeros32 = jnp.zeros((ZCHUNK, 32), jnp.float32)
    onehot = (batch_idx[:, None]
              == jnp.arange(BATCHES, dtype=jnp.int32)[None, :]
              ).astype(jnp.bfloat16)

    eps1v = jnp.reshape(eps1, (1, 1))
    eps2v = jnp.reshape(eps2, (1, 1))
    row = lambda a: jnp.reshape(a, (1, -1))

    p1 = _agg1_sc(nodes16, e3, zeros16)
    x1a, x1b = _conv1_dense(nodes16, p1, W1p, row(b1), eps1v,
                            row(g1), row(be1), W2, row(b2))

    agg2 = _agg2_sc(x1a, x1b, e3, zeros32)
    out = _conv2_tail(x1a, x1b, agg2, W3, row(b3), eps2v, row(g2), row(be2),
                      W4, row(b4), onehot, W5, row(b5))
    return out


# speedup vs baseline: 1.0924x; 1.0221x over previous
import functools

import jax
import jax.numpy as jnp
from jax import lax
from jax.experimental import pallas as pl
from jax.experimental.pallas import tpu as pltpu
from jax.experimental.pallas import tpu_sc as plsc

N = 50000
E = 800000
BATCHES = 16

NUM_CORES = 2
NUM_SUBCORES = 16
SUB = 100
KROWS = 10
EROWS = E // SUB
G = EROWS // (NUM_SUBCORES * KROWS)

ZCHUNK = 3128
ZLAST = N - (NUM_SUBCORES - 1) * ZCHUNK

BLK = 2000
T = N // BLK

_mesh = functools.partial(
    plsc.VectorSubcoreMesh,
    core_axis_name="c", subcore_axis_name="s",
    num_cores=NUM_CORES, num_subcores=NUM_SUBCORES,
)


NBUF = 8
PF = 4


def _sc_edge_loop(x_hbm, e_hbm, acc, idx_bufs, rows_bufs,
                  gsem, ssem, isem, row0, n_outer, col0=None, ncol=None,
                  nbuf=NBUF, pf=PF):
    (sA, dA), (sB, dB) = idx_bufs

    def fire_idx(i, sbuf, dbuf):
        r = row0 + i * KROWS
        pltpu.async_copy(e_hbm.at[0, pl.ds(r, KROWS)], sbuf, isem)
        pltpu.async_copy(e_hbm.at[1, pl.ds(r, KROWS)], dbuf, isem)

    def wait_idx(sbuf, dbuf):
        pltpu.make_async_copy(e_hbm.at[0, pl.ds(0, KROWS)], sbuf, isem).wait()
        pltpu.make_async_copy(e_hbm.at[1, pl.ds(0, KROWS)], dbuf, isem).wait()

    def gsrc(idx):
        if col0 is None:
            return x_hbm.at[idx]
        return x_hbm.at[idx, pl.ds(col0, ncol)]

    def process(idx_s, idx_d):
        gd = {}
        sd = {}
        for j in range(pf):
            gd[j] = pltpu.async_copy(gsrc(idx_s.at[j]),
                                     rows_bufs[j % nbuf], gsem)
        for j in range(KROWS):
            nj = j + pf
            if nj < KROWS:
                if nj >= nbuf:
                    sd[nj - nbuf].wait()
                gd[nj] = pltpu.async_copy(gsrc(idx_s.at[nj]),
                                          rows_bufs[nj % nbuf], gsem)
            gd[j].wait()
            sd[j] = pltpu.async_copy(rows_bufs[j % nbuf],
                                     acc.at[idx_d.at[j]], ssem, add=True)
        for j in range(max(0, KROWS - nbuf), KROWS):
            sd[j].wait()

    fire_idx(0, sA, dA)

    def outer(io, _):
        i0 = 2 * io

        @pl.when(i0 + 1 < n_outer)
        def _():
            fire_idx(i0 + 1, sB, dB)

        wait_idx(sA, dA)
        process(sA, dA)

        @pl.when(i0 + 2 < n_outer)
        def _():
            fire_idx(i0 + 2, sA, dA)

        @pl.when(i0 + 1 < n_outer)
        def _():
            wait_idx(sB, dB)
            process(sB, dB)

        return 0

    lax.fori_loop(0, (n_outer + 1) // 2, outer, 0, unroll=False)


def _zero_acc(z_hbm, acc, s):
    @pl.when(s < NUM_SUBCORES - 1)
    def _():
        pltpu.sync_copy(z_hbm, acc.at[pl.ds(s * ZCHUNK, ZCHUNK)])

    @pl.when(s == NUM_SUBCORES - 1)
    def _():
        pltpu.sync_copy(z_hbm.at[pl.ds(0, ZLAST)],
                        acc.at[pl.ds((NUM_SUBCORES - 1) * ZCHUNK, ZLAST)])


def _write_out(acc, out_hbm, c, s, dc):
    @pl.when(s < NUM_SUBCORES - 1)
    def _():
        pltpu.sync_copy(acc.at[pl.ds(s * ZCHUNK, ZCHUNK)],
                        out_hbm.at[pl.ds(s * ZCHUNK, ZCHUNK), pl.ds(dc * c, dc)])

    @pl.when(s == NUM_SUBCORES - 1)
    def _():
        pltpu.sync_copy(
            acc.at[pl.ds((NUM_SUBCORES - 1) * ZCHUNK, ZLAST)],
            out_hbm.at[pl.ds((NUM_SUBCORES - 1) * ZCHUNK, ZLAST),
                       pl.ds(dc * c, dc)])


def _agg1_sc(nodes16, e3, zeros16):
    rows_per_core = EROWS // NUM_CORES
    rows_per_tile = rows_per_core // NUM_SUBCORES
    n_outer = rows_per_tile // KROWS

    def body(nodes_hbm, e_hbm, z_hbm, out_hbm,
             acc, isa, ida, isb, idb, r0, r1, r2, r3, r4, r5, r6, r7,
             r8, r9, gsem, ssem, isem):
        c = lax.axis_index("c")
        s = lax.axis_index("s")
        _zero_acc(z_hbm, acc, s)
        plsc.subcore_barrier()
        row0 = c * rows_per_core + s * rows_per_tile
        _sc_edge_loop(nodes_hbm, e_hbm, acc, [(isa, ida), (isb, idb)],
                      [r0, r1, r2, r3, r4, r5, r6, r7, r8, r9],
                      gsem, ssem, isem, row0, n_outer, nbuf=10, pf=10)
        plsc.subcore_barrier()
        _write_out(acc, out_hbm, c, s, 16)

    k = pl.kernel(
        body,
        out_type=jax.ShapeDtypeStruct((N, 128), jnp.float32),
        mesh=_mesh(),
        compiler_params=pltpu.CompilerParams(use_tc_tiling_on_sc=False),
        scratch_types=[
            pltpu.VMEM_SHARED((N, 16), jnp.float32),
            pltpu.VMEM((KROWS, SUB), jnp.int32),
            pltpu.VMEM((KROWS, SUB), jnp.int32),
            pltpu.VMEM((KROWS, SUB), jnp.int32),
            pltpu.VMEM((KROWS, SUB), jnp.int32),
            pltpu.VMEM((SUB, 16), jnp.float32),
            pltpu.VMEM((SUB, 16), jnp.float32),
            pltpu.VMEM((SUB, 16), jnp.float32),
            pltpu.VMEM((SUB, 16), jnp.float32),
            pltpu.VMEM((SUB, 16), jnp.float32),
            pltpu.VMEM((SUB, 16), jnp.float32),
            pltpu.VMEM((SUB, 16), jnp.float32),
            pltpu.VMEM((SUB, 16), jnp.float32),
            pltpu.VMEM((SUB, 16), jnp.float32),
            pltpu.VMEM((SUB, 16), jnp.float32),
            pltpu.SemaphoreType.DMA,
            pltpu.SemaphoreType.DMA,
            pltpu.SemaphoreType.DMA,
        ],
    )
    return k(nodes16, e3, zeros16)


def _agg2_sc(x1a, x1b, e3, zeros32):
    rows_per_tile = EROWS // NUM_SUBCORES

    def body(xa_hbm, xb_hbm, e_hbm, z_hbm, out_hbm,
             acc, isa, ida, isb, idb, r0, r1, r2, r3, r4, r5, r6, r7,
             gsem, ssem, isem):
        c = lax.axis_index("c")
        s = lax.axis_index("s")
        _zero_acc(z_hbm, acc, s)
        plsc.subcore_barrier()
        ibufs = [(isa, ida), (isb, idb)]
        bufs = [r0, r1, r2, r3, r4, r5, r6, r7]
        row0 = s * rows_per_tile

        @pl.when(c == 0)
        def _():
            _sc_edge_loop(xa_hbm, e_hbm, acc, ibufs, bufs,
                          gsem, ssem, isem, row0, G)

        @pl.when(c == 1)
        def _():
            _sc_edge_loop(xb_hbm, e_hbm, acc, ibufs, bufs,
                          gsem, ssem, isem, row0, G)

        plsc.subcore_barrier()
        _write_out(acc, out_hbm, c, s, 32)

    k = pl.kernel(
        body,
        out_type=jax.ShapeDtypeStruct((N, 128), jnp.float32),
        mesh=_mesh(),
        compiler_params=pltpu.CompilerParams(use_tc_tiling_on_sc=False),
        scratch_types=[
            pltpu.VMEM_SHARED((N, 32), jnp.float32),
            pltpu.VMEM((KROWS, SUB), jnp.int32),
            pltpu.VMEM((KROWS, SUB), jnp.int32),
            pltpu.VMEM((KROWS, SUB), jnp.int32),
            pltpu.VMEM((KROWS, SUB), jnp.int32),
            pltpu.VMEM((SUB, 32), jnp.float32),
            pltpu.VMEM((SUB, 32), jnp.float32),
            pltpu.VMEM((SUB, 32), jnp.float32),
            pltpu.VMEM((SUB, 32), jnp.float32),
            pltpu.VMEM((SUB, 32), jnp.float32),
            pltpu.VMEM((SUB, 32), jnp.float32),
            pltpu.VMEM((SUB, 32), jnp.float32),
            pltpu.VMEM((SUB, 32), jnp.float32),
            pltpu.SemaphoreType.DMA,
            pltpu.SemaphoreType.DMA,
            pltpu.SemaphoreType.DMA,
        ],
    )
    return k(x1a, x1b, e3, zeros32)


def _gelu(x):
    return x * 0.5 * (1.0 + lax.erf(x * (2.0 ** -0.5)))


def _conv1_dense_kernel(x_ref, p_ref, w1_ref, b1_ref, eps_ref, g_ref,
                        be_ref, w2_ref, b2_ref, oa_ref, ob_ref,
                        h_all, ssum, ssq):
    p = pl.program_id(0)
    i = pl.program_id(1)

    @pl.when(p == 0)
    def _():
        x = x_ref[...]
        pp = p_ref[...]
        agg = pp[:, :16] + pp[:, 16:32]
        out = (1.0 + eps_ref[0, 0]) * x + agg
        h = jnp.dot(out, w1_ref[...],
                    preferred_element_type=jnp.float32) + b1_ref[...]
        h_all[pl.ds(i * BLK, BLK), :] = h
        s = jnp.sum(h, axis=0, keepdims=True)
        q = jnp.sum(h * h, axis=0, keepdims=True)

        @pl.when(i == 0)
        def _():
            ssum[...] = s
            ssq[...] = q

        @pl.when(i > 0)
        def _():
            ssum[...] += s
            ssq[...] += q

    @pl.when(p == 1)
    def _():
        m = ssum[...] / N
        v = ssq[...] / N - m * m
        scale = g_ref[...] * lax.rsqrt(v + 1e-5)
        h = h_all[pl.ds(i * BLK, BLK), :]
        xn = (h - m) * scale + be_ref[...]
        ge = _gelu(xn)
        y = jnp.dot(ge, w2_ref[...],
                    preferred_element_type=jnp.float32) + b2_ref[...]
        oa_ref[...] = y[:, :32]
        ob_ref[...] = y[:, 32:]


def _conv1_dense(nodes16, partials, w1, b1, eps, g, be, w2, b2):
    blk = lambda p, i: (i * (1 - p), 0)
    return pl.pallas_call(
        _conv1_dense_kernel,
        grid=(2, T),
        in_specs=[
            pl.BlockSpec((BLK, 16), blk),
            pl.BlockSpec((BLK, 128), lambda p, i: (i * (1 - p), 0)),
            pl.BlockSpec((16, 64), lambda p, i: (0, 0)),
            pl.BlockSpec((1, 64), lambda p, i: (0, 0)),
            pl.BlockSpec((1, 1), lambda p, i: (0, 0)),
            pl.BlockSpec((1, 64), lambda p, i: (0, 0)),
            pl.BlockSpec((1, 64), lambda p, i: (0, 0)),
            pl.BlockSpec((64, 64), lambda p, i: (0, 0)),
            pl.BlockSpec((1, 64), lambda p, i: (0, 0)),
        ],
        out_specs=[
            pl.BlockSpec((BLK, 32), lambda p, i: (i * p, 0)),
            pl.BlockSpec((BLK, 32), lambda p, i: (i * p, 0)),
        ],
        out_shape=[
            jax.ShapeDtypeStruct((N, 32), jnp.float32),
            jax.ShapeDtypeStruct((N, 32), jnp.float32),
        ],
        scratch_shapes=[
            pltpu.VMEM((N, 64), jnp.float32),
            pltpu.VMEM((1, 64), jnp.float32),
            pltpu.VMEM((1, 64), jnp.float32),
        ],
    )(nodes16, partials, w1, b1, eps, g, be, w2, b2)


def _conv2_tail_kernel(xa_ref, xb_ref, p_ref, w3_ref, b3_ref, eps_ref,
                       g_ref, be_ref, w4_ref, b4_ref, oh_ref, w5_ref,
                       b5_ref, out_ref, h_all, ssum, ssq, pacc):
    p = pl.program_id(0)
    i = pl.program_id(1)

    @pl.when(p == 0)
    def _():
        x = jnp.concatenate([xa_ref[...], xb_ref[...]], axis=1)
        agg = p_ref[:, :64]
        out = (1.0 + eps_ref[0, 0]) * x + agg
        h = jnp.dot(out, w3_ref[...],
                    preferred_element_type=jnp.float32) + b3_ref[...]
        h_all[pl.ds(i * BLK, BLK), :] = h
        s = jnp.sum(h, axis=0, keepdims=True)
        q = jnp.sum(h * h, axis=0, keepdims=True)

        @pl.when(i == 0)
        def _():
            ssum[...] = s
            ssq[...] = q

        @pl.when(i > 0)
        def _():
            ssum[...] += s
            ssq[...] += q

    @pl.when(p == 1)
    def _():
        m = ssum[...] / N
        v = ssq[...] / N - m * m
        scale = g_ref[...] * lax.rsqrt(v + 1e-5)
        h = h_all[pl.ds(i * BLK, BLK), :]
        xn = (h - m) * scale + be_ref[...]
        ge = _gelu(xn)
        x2 = jnp.dot(ge, w4_ref[...],
                     preferred_element_type=jnp.float32) + b4_ref[...]
        part = lax.dot_general(oh_ref[...].astype(jnp.float32), x2,
                               (((0,), (0,)), ((), ())),
                               preferred_element_type=jnp.float32)

        @pl.when(i == 0)
        def _():
            pacc[...] = part

        @pl.when(i > 0)
        def _():
            pacc[...] += part

        @pl.when(i == T - 1)
        def _():
            o = jnp.dot(pacc[...], w5_ref[...],
                        preferred_element_type=jnp.float32) + b5_ref[...]
            out_ref[...] = _gelu(o)


def _conv2_tail(x1a, x1b, agg, w3, b3, eps, g, be, w4, b4, onehot, w5, b5):
    return pl.pallas_call(
        _conv2_tail_kernel,
        grid=(2, T),
        in_specs=[
            pl.BlockSpec((BLK, 32), lambda p, i: (i * (1 - p), 0)),
            pl.BlockSpec((BLK, 32), lambda p, i: (i * (1 - p), 0)),
            pl.BlockSpec((BLK, 128), lambda p, i: (i * (1 - p), 0)),
            pl.BlockSpec((64, 128), lambda p, i: (0, 0)),
            pl.BlockSpec((1, 128), lambda p, i: (0, 0)),
            pl.BlockSpec((1, 1), lambda p, i: (0, 0)),
            pl.BlockSpec((1, 128), lambda p, i: (0, 0)),
            pl.BlockSpec((1, 128), lambda p, i: (0, 0)),
            pl.BlockSpec((128, 128), lambda p, i: (0, 0)),
            pl.BlockSpec((1, 128), lambda p, i: (0, 0)),
            pl.BlockSpec((BLK, BATCHES), lambda p, i: (i * p, 0)),
            pl.BlockSpec((128, 128), lambda p, i: (0, 0)),
            pl.BlockSpec((1, 128), lambda p, i: (0, 0)),
        ],
        out_specs=pl.BlockSpec((BATCHES, 128), lambda p, i: (0, 0)),
        out_shape=jax.ShapeDtypeStruct((BATCHES, 128), jnp.float32),
        scratch_shapes=[
            pltpu.VMEM((N, 128), jnp.float32),
            pltpu.VMEM((1, 128), jnp.float32),
            pltpu.VMEM((1, 128), jnp.float32),
            pltpu.VMEM((BATCHES, 128), jnp.float32),
        ],
    )(x1a, x1b, agg, w3, b3, eps, g, be, w4, b4, onehot, w5, b5)


def kernel(nodes, edge_idx, batch_idx, B, eps1, W1, b1, g1, be1, W2, b2,
           eps2, W3, b3, g2, be2, W4, b4, W5, b5):
    nodes16 = jnp.pad(nodes, ((0, 0), (0, 11)))
    W1p = jnp.pad(W1, ((0, 11), (0, 0)))
    e3 = edge_idx.reshape(2, EROWS, SUB)
    zeros16 = jnp.zeros((ZCHUNK, 16), jnp.float32)
    zeros32 = jnp.zeros((ZCHUNK, 32), jnp.float32)
    onehot = (batch_idx[:, None]
              == jnp.arange(BATCHES, dtype=jnp.int32)[None, :]
              ).astype(jnp.bfloat16)

    eps1v = jnp.reshape(eps1, (1, 1))
    eps2v = jnp.reshape(eps2, (1, 1))
    row = lambda a: jnp.reshape(a, (1, -1))

    p1 = _agg1_sc(nodes16, e3, zeros16)
    x1a, x1b = _conv1_dense(nodes16, p1, W1p, row(b1), eps1v,
                            row(g1), row(be1), W2, row(b2))

    agg2 = _agg2_sc(x1a, x1b, e3, zeros32)
    out = _conv2_tail(x1a, x1b, agg2, W3, row(b3), eps2v, row(g2), row(be2),
                      W4, row(b4), onehot, W5, row(b5))
    return out


# speedup vs baseline: 1.0974x; 1.0046x over previous
import functools

import jax
import jax.numpy as jnp
from jax import lax
from jax.experimental import pallas as pl
from jax.experimental.pallas import tpu as pltpu
from jax.experimental.pallas import tpu_sc as plsc

N = 50000
E = 800000
BATCHES = 16

NUM_CORES = 2
NUM_SUBCORES = 16
SUB = 100
KROWS = 10
EROWS = E // SUB
G = EROWS // (NUM_SUBCORES * KROWS)

ZCHUNK = 3128
ZLAST = N - (NUM_SUBCORES - 1) * ZCHUNK

BLK = 2000
T = N // BLK

_mesh = functools.partial(
    plsc.VectorSubcoreMesh,
    core_axis_name="c", subcore_axis_name="s",
    num_cores=NUM_CORES, num_subcores=NUM_SUBCORES,
)


NBUF = 8
PF = 7


def _sc_edge_loop(x_hbm, e_hbm, acc, idx_bufs, rows_bufs,
                  gsem, ssem, isem, row0, n_outer, col0=None, ncol=None,
                  nbuf=NBUF, pf=PF):
    (sA, dA), (sB, dB) = idx_bufs

    def fire_idx(i, sbuf, dbuf):
        r = row0 + i * KROWS
        pltpu.async_copy(e_hbm.at[0, pl.ds(r, KROWS)], sbuf, isem)
        pltpu.async_copy(e_hbm.at[1, pl.ds(r, KROWS)], dbuf, isem)

    def wait_idx(sbuf, dbuf):
        pltpu.make_async_copy(e_hbm.at[0, pl.ds(0, KROWS)], sbuf, isem).wait()
        pltpu.make_async_copy(e_hbm.at[1, pl.ds(0, KROWS)], dbuf, isem).wait()

    def gsrc(idx):
        if col0 is None:
            return x_hbm.at[idx]
        return x_hbm.at[idx, pl.ds(col0, ncol)]

    def process(idx_s, idx_d):
        gd = {}
        sd = {}
        for j in range(pf):
            gd[j] = pltpu.async_copy(gsrc(idx_s.at[j]),
                                     rows_bufs[j % nbuf], gsem)
        for j in range(KROWS):
            nj = j + pf
            if nj < KROWS:
                if nj >= nbuf:
                    sd[nj - nbuf].wait()
                gd[nj] = pltpu.async_copy(gsrc(idx_s.at[nj]),
                                          rows_bufs[nj % nbuf], gsem)
            gd[j].wait()
            sd[j] = pltpu.async_copy(rows_bufs[j % nbuf],
                                     acc.at[idx_d.at[j]], ssem, add=True)
        for j in range(max(0, KROWS - nbuf), KROWS):
            sd[j].wait()

    fire_idx(0, sA, dA)

    def outer(io, _):
        i0 = 2 * io

        @pl.when(i0 + 1 < n_outer)
        def _():
            fire_idx(i0 + 1, sB, dB)

        wait_idx(sA, dA)
        process(sA, dA)

        @pl.when(i0 + 2 < n_outer)
        def _():
            fire_idx(i0 + 2, sA, dA)

        @pl.when(i0 + 1 < n_outer)
        def _():
            wait_idx(sB, dB)
            process(sB, dB)

        return 0

    lax.fori_loop(0, (n_outer + 1) // 2, outer, 0, unroll=False)


def _zero_acc(z_hbm, acc, s):
    @pl.when(s < NUM_SUBCORES - 1)
    def _():
        pltpu.sync_copy(z_hbm, acc.at[pl.ds(s * ZCHUNK, ZCHUNK)])

    @pl.when(s == NUM_SUBCORES - 1)
    def _():
        pltpu.sync_copy(z_hbm.at[pl.ds(0, ZLAST)],
                        acc.at[pl.ds((NUM_SUBCORES - 1) * ZCHUNK, ZLAST)])


def _write_out(acc, out_hbm, c, s, dc):
    @pl.when(s < NUM_SUBCORES - 1)
    def _():
        pltpu.sync_copy(acc.at[pl.ds(s * ZCHUNK, ZCHUNK)],
                        out_hbm.at[pl.ds(s * ZCHUNK, ZCHUNK), pl.ds(dc * c, dc)])

    @pl.when(s == NUM_SUBCORES - 1)
    def _():
        pltpu.sync_copy(
            acc.at[pl.ds((NUM_SUBCORES - 1) * ZCHUNK, ZLAST)],
            out_hbm.at[pl.ds((NUM_SUBCORES - 1) * ZCHUNK, ZLAST),
                       pl.ds(dc * c, dc)])


def _agg1_sc(nodes16, e3, zeros16):
    rows_per_core = EROWS // NUM_CORES
    rows_per_tile = rows_per_core // NUM_SUBCORES
    n_outer = rows_per_tile // KROWS

    def body(nodes_hbm, e_hbm, z_hbm, out_hbm,
             acc, isa, ida, isb, idb, r0, r1, r2, r3, r4, r5, r6, r7,
             r8, r9, gsem, ssem, isem):
        c = lax.axis_index("c")
        s = lax.axis_index("s")
        _zero_acc(z_hbm, acc, s)
        plsc.subcore_barrier()
        row0 = c * rows_per_core + s * rows_per_tile
        _sc_edge_loop(nodes_hbm, e_hbm, acc, [(isa, ida), (isb, idb)],
                      [r0, r1, r2, r3, r4, r5, r6, r7, r8, r9],
                      gsem, ssem, isem, row0, n_outer, nbuf=10, pf=10)
        plsc.subcore_barrier()
        _write_out(acc, out_hbm, c, s, 16)

    k = pl.kernel(
        body,
        out_type=jax.ShapeDtypeStruct((N, 128), jnp.float32),
        mesh=_mesh(),
        compiler_params=pltpu.CompilerParams(use_tc_tiling_on_sc=False),
        scratch_types=[
            pltpu.VMEM_SHARED((N, 16), jnp.float32),
            pltpu.VMEM((KROWS, SUB), jnp.int32),
            pltpu.VMEM((KROWS, SUB), jnp.int32),
            pltpu.VMEM((KROWS, SUB), jnp.int32),
            pltpu.VMEM((KROWS, SUB), jnp.int32),
            pltpu.VMEM((SUB, 16), jnp.float32),
            pltpu.VMEM((SUB, 16), jnp.float32),
            pltpu.VMEM((SUB, 16), jnp.float32),
            pltpu.VMEM((SUB, 16), jnp.float32),
            pltpu.VMEM((SUB, 16), jnp.float32),
            pltpu.VMEM((SUB, 16), jnp.float32),
            pltpu.VMEM((SUB, 16), jnp.float32),
            pltpu.VMEM((SUB, 16), jnp.float32),
            pltpu.VMEM((SUB, 16), jnp.float32),
            pltpu.VMEM((SUB, 16), jnp.float32),
            pltpu.SemaphoreType.DMA,
            pltpu.SemaphoreType.DMA,
            pltpu.SemaphoreType.DMA,
        ],
    )
    return k(nodes16, e3, zeros16)


def _agg2_sc(x1a, x1b, e3, zeros32):
    rows_per_tile = EROWS // NUM_SUBCORES

    def body(xa_hbm, xb_hbm, e_hbm, z_hbm, out_hbm,
             acc, isa, ida, isb, idb, r0, r1, r2, r3, r4, r5, r6, r7,
             gsem, ssem, isem):
        c = lax.axis_index("c")
        s = lax.axis_index("s")
        _zero_acc(z_hbm, acc, s)
        plsc.subcore_barrier()
        ibufs = [(isa, ida), (isb, idb)]
        bufs = [r0, r1, r2, r3, r4, r5, r6, r7]
        row0 = s * rows_per_tile

        @pl.when(c == 0)
        def _():
            _sc_edge_loop(xa_hbm, e_hbm, acc, ibufs, bufs,
                          gsem, ssem, isem, row0, G)

        @pl.when(c == 1)
        def _():
            _sc_edge_loop(xb_hbm, e_hbm, acc, ibufs, bufs,
                          gsem, ssem, isem, row0, G)

        plsc.subcore_barrier()
        _write_out(acc, out_hbm, c, s, 32)

    k = pl.kernel(
        body,
        out_type=jax.ShapeDtypeStruct((N, 128), jnp.float32),
        mesh=_mesh(),
        compiler_params=pltpu.CompilerParams(use_tc_tiling_on_sc=False),
        scratch_types=[
            pltpu.VMEM_SHARED((N, 32), jnp.float32),
            pltpu.VMEM((KROWS, SUB), jnp.int32),
            pltpu.VMEM((KROWS, SUB), jnp.int32),
            pltpu.VMEM((KROWS, SUB), jnp.int32),
            pltpu.VMEM((KROWS, SUB), jnp.int32),
            pltpu.VMEM((SUB, 32), jnp.float32),
            pltpu.VMEM((SUB, 32), jnp.float32),
            pltpu.VMEM((SUB, 32), jnp.float32),
            pltpu.VMEM((SUB, 32), jnp.float32),
            pltpu.VMEM((SUB, 32), jnp.float32),
            pltpu.VMEM((SUB, 32), jnp.float32),
            pltpu.VMEM((SUB, 32), jnp.float32),
            pltpu.VMEM((SUB, 32), jnp.float32),
            pltpu.SemaphoreType.DMA,
            pltpu.SemaphoreType.DMA,
            pltpu.SemaphoreType.DMA,
        ],
    )
    return k(x1a, x1b, e3, zeros32)


def _gelu(x):
    return x * 0.5 * (1.0 + lax.erf(x * (2.0 ** -0.5)))


def _conv1_dense_kernel(x_ref, p_ref, w1_ref, b1_ref, eps_ref, g_ref,
                        be_ref, w2_ref, b2_ref, oa_ref, ob_ref,
                        h_all, ssum, ssq):
    p = pl.program_id(0)
    i = pl.program_id(1)

    @pl.when(p == 0)
    def _():
        x = x_ref[...]
        pp = p_ref[...]
        agg = pp[:, :16] + pp[:, 16:32]
        out = (1.0 + eps_ref[0, 0]) * x + agg
        h = jnp.dot(out, w1_ref[...],
                    preferred_element_type=jnp.float32) + b1_ref[...]
        h_all[pl.ds(i * BLK, BLK), :] = h
        s = jnp.sum(h, axis=0, keepdims=True)
        q = jnp.sum(h * h, axis=0, keepdims=True)

        @pl.when(i == 0)
        def _():
            ssum[...] = s
            ssq[...] = q

        @pl.when(i > 0)
        def _():
            ssum[...] += s
            ssq[...] += q

    @pl.when(p == 1)
    def _():
        m = ssum[...] / N
        v = ssq[...] / N - m * m
        scale = g_ref[...] * lax.rsqrt(v + 1e-5)
        h = h_all[pl.ds(i * BLK, BLK), :]
        xn = (h - m) * scale + be_ref[...]
        ge = _gelu(xn)
        y = jnp.dot(ge, w2_ref[...],
                    preferred_element_type=jnp.float32) + b2_ref[...]
        oa_ref[...] = y[:, :32]
        ob_ref[...] = y[:, 32:]


def _conv1_dense(nodes16, partials, w1, b1, eps, g, be, w2, b2):
    blk = lambda p, i: (i * (1 - p), 0)
    return pl.pallas_call(
        _conv1_dense_kernel,
        grid=(2, T),
        in_specs=[
            pl.BlockSpec((BLK, 16), blk),
            pl.BlockSpec((BLK, 128), lambda p, i: (i * (1 - p), 0)),
            pl.BlockSpec((16, 64), lambda p, i: (0, 0)),
            pl.BlockSpec((1, 64), lambda p, i: (0, 0)),
            pl.BlockSpec((1, 1), lambda p, i: (0, 0)),
            pl.BlockSpec((1, 64), lambda p, i: (0, 0)),
            pl.BlockSpec((1, 64), lambda p, i: (0, 0)),
            pl.BlockSpec((64, 64), lambda p, i: (0, 0)),
            pl.BlockSpec((1, 64), lambda p, i: (0, 0)),
        ],
        out_specs=[
            pl.BlockSpec((BLK, 32), lambda p, i: (i * p, 0)),
            pl.BlockSpec((BLK, 32), lambda p, i: (i * p, 0)),
        ],
        out_shape=[
            jax.ShapeDtypeStruct((N, 32), jnp.float32),
            jax.ShapeDtypeStruct((N, 32), jnp.float32),
        ],
        scratch_shapes=[
            pltpu.VMEM((N, 64), jnp.float32),
            pltpu.VMEM((1, 64), jnp.float32),
            pltpu.VMEM((1, 64), jnp.float32),
        ],
    )(nodes16, partials, w1, b1, eps, g, be, w2, b2)


def _conv2_tail_kernel(xa_ref, xb_ref, p_ref, w3_ref, b3_ref, eps_ref,
                       g_ref, be_ref, w4_ref, b4_ref, oh_ref, w5_ref,
                       b5_ref, out_ref, h_all, ssum, ssq, pacc):
    p = pl.program_id(0)
    i = pl.program_id(1)

    @pl.when(p == 0)
    def _():
        x = jnp.concatenate([xa_ref[...], xb_ref[...]], axis=1)
        agg = p_ref[:, :64]
        out = (1.0 + eps_ref[0, 0]) * x + agg
        h = jnp.dot(out, w3_ref[...],
                    preferred_element_type=jnp.float32) + b3_ref[...]
        h_all[pl.ds(i * BLK, BLK), :] = h
        s = jnp.sum(h, axis=0, keepdims=True)
        q = jnp.sum(h * h, axis=0, keepdims=True)

        @pl.when(i == 0)
        def _():
            ssum[...] = s
            ssq[...] = q

        @pl.when(i > 0)
        def _():
            ssum[...] += s
            ssq[...] += q

    @pl.when(p == 1)
    def _():
        m = ssum[...] / N
        v = ssq[...] / N - m * m
        scale = g_ref[...] * lax.rsqrt(v + 1e-5)
        h = h_all[pl.ds(i * BLK, BLK), :]
        xn = (h - m) * scale + be_ref[...]
        ge = _gelu(xn)
        x2 = jnp.dot(ge, w4_ref[...],
                     preferred_element_type=jnp.float32) + b4_ref[...]
        part = lax.dot_general(oh_ref[...].astype(jnp.float32), x2,
                               (((0,), (0,)), ((), ())),
                               preferred_element_type=jnp.float32)

        @pl.when(i == 0)
        def _():
            pacc[...] = part

        @pl.when(i > 0)
        def _():
            pacc[...] += part

        @pl.when(i == T - 1)
        def _():
            o = jnp.dot(pacc[...], w5_ref[...],
                        preferred_element_type=jnp.float32) + b5_ref[...]
            out_ref[...] = _gelu(o)


def _conv2_tail(x1a, x1b, agg, w3, b3, eps, g, be, w4, b4, onehot, w5, b5):
    return pl.pallas_call(
        _conv2_tail_kernel,
        grid=(2, T),
        in_specs=[
            pl.BlockSpec((BLK, 32), lambda p, i: (i * (1 - p), 0)),
            pl.BlockSpec((BLK, 32), lambda p, i: (i * (1 - p), 0)),
            pl.BlockSpec((BLK, 128), lambda p, i: (i * (1 - p), 0)),
            pl.BlockSpec((64, 128), lambda p, i: (0, 0)),
            pl.BlockSpec((1, 128), lambda p, i: (0, 0)),
            pl.BlockSpec((1, 1), lambda p, i: (0, 0)),
            pl.BlockSpec((1, 128), lambda p, i: (0, 0)),
            pl.BlockSpec((1, 128), lambda p, i: (0, 0)),
            pl.BlockSpec((128, 128), lambda p, i: (0, 0)),
            pl.BlockSpec((1, 128), lambda p, i: (0, 0)),
            pl.BlockSpec((BLK, BATCHES), lambda p, i: (i * p, 0)),
            pl.BlockSpec((128, 128), lambda p, i: (0, 0)),
            pl.BlockSpec((1, 128), lambda p, i: (0, 0)),
        ],
        out_specs=pl.BlockSpec((BATCHES, 128), lambda p, i: (0, 0)),
        out_shape=jax.ShapeDtypeStruct((BATCHES, 128), jnp.float32),
        scratch_shapes=[
            pltpu.VMEM((N, 128), jnp.float32),
            pltpu.VMEM((1, 128), jnp.float32),
            pltpu.VMEM((1, 128), jnp.float32),
            pltpu.VMEM((BATCHES, 128), jnp.float32),
        ],
    )(x1a, x1b, agg, w3, b3, eps, g, be, w4, b4, onehot, w5, b5)


def kernel(nodes, edge_idx, batch_idx, B, eps1, W1, b1, g1, be1, W2, b2,
           eps2, W3, b3, g2, be2, W4, b4, W5, b5):
    nodes16 = jnp.pad(nodes, ((0, 0), (0, 11)))
    W1p = jnp.pad(W1, ((0, 11), (0, 0)))
    e3 = edge_idx.reshape(2, EROWS, SUB)
    zeros16 = jnp.zeros((ZCHUNK, 16), jnp.float32)
    zeros32 = jnp.zeros((ZCHUNK, 32), jnp.float32)
    onehot = (batch_idx[:, None]
              == jnp.arange(BATCHES, dtype=jnp.int32)[None, :]
              ).astype(jnp.bfloat16)

    eps1v = jnp.reshape(eps1, (1, 1))
    eps2v = jnp.reshape(eps2, (1, 1))
    row = lambda a: jnp.reshape(a, (1, -1))

    p1 = _agg1_sc(nodes16, e3, zeros16)
    x1a, x1b = _conv1_dense(nodes16, p1, W1p, row(b1), eps1v,
                            row(g1), row(be1), W2, row(b2))

    agg2 = _agg2_sc(x1a, x1b, e3, zeros32)
    out = _conv2_tail(x1a, x1b, agg2, W3, row(b3), eps2v, row(g2), row(be2),
                      W4, row(b4), onehot, W5, row(b5))
    return out
